# baseline (device time: 58844 ns/iter reference)
import jax
import jax.numpy as jnp
from jax import lax
from jax.experimental import pallas as pl
from jax.experimental.pallas import tpu as pltpu

N_DEV = 4
M_PER = 64
M = N_DEV * M_PER
D = 512
H_PER = 1024


def kernel(x, Win0, Wout0, Win1, Wout1, Win2, Wout2):
    def body(x_ref, win0_ref, wout0_ref, win1_ref, wout1_ref,
             win2_ref, wout2_ref, out_ref,
             xfull, zbuf, comm, send_sems, recv_sems):
        p = lax.axis_index("i")
        left = lax.rem(p + N_DEV - 1, N_DEV)
        right = lax.rem(p + 1, N_DEV)

        barrier_sem = pltpu.get_barrier_semaphore()
        for nbr in (left, right):
            pl.semaphore_signal(
                barrier_sem, inc=1,
                device_id=(nbr,), device_id_type=pl.DeviceIdType.MESH,
            )
        pl.semaphore_wait(barrier_sem, 2)

        def ring_hop(g):
            s_slot = g % 2
            r_slot = (g + 1) % 2
            rdma = pltpu.make_async_remote_copy(
                src_ref=comm.at[s_slot],
                dst_ref=comm.at[r_slot],
                send_sem=send_sems.at[s_slot],
                recv_sem=recv_sems.at[r_slot],
                device_id=(right,),
                device_id_type=pl.DeviceIdType.MESH,
            )
            rdma.start()
            rdma.wait()
            return r_slot

        myx = x_ref[:, :].astype(jnp.bfloat16)
        xfull[pl.ds(p * M_PER, M_PER), :] = myx
        comm[0, :, :] = myx
        g = 0
        for h in range(N_DEV - 1):
            r = ring_hop(g)
            g += 1
            origin = lax.rem(p + N_DEV - h - 1, N_DEV)
            xfull[pl.ds(origin * M_PER, M_PER), :] = comm[r, :, :]

        def layer(win_ref, wout_ref, g, last):
            X = xfull[:, :]
            W1 = win_ref[:, :].astype(jnp.bfloat16)
            W2 = wout_ref[:, :].astype(jnp.bfloat16)
            h1 = jnp.dot(X, W1, preferred_element_type=jnp.float32)
            h1 = jnp.maximum(h1, 0.0).astype(jnp.bfloat16)
            z = jnp.dot(h1, W2, preferred_element_type=jnp.float32)
            zbuf[:, :] = z.astype(jnp.bfloat16)

            first = lax.rem(p + N_DEV - 1, N_DEV)
            comm[g % 2, :, :] = zbuf[pl.ds(first * M_PER, M_PER), :]
            for s in range(N_DEV - 1):
                r = ring_hop(g)
                g += 1
                c = lax.rem(p + 2 * N_DEV - s - 2, N_DEV)
                comm[r, :, :] = (
                    comm[r, :, :] + zbuf[pl.ds(c * M_PER, M_PER), :]
                )

            if last:
                out_ref[:, :] = comm[r, :, :]
            else:
                xfull[pl.ds(p * M_PER, M_PER), :] = comm[r, :, :]
                for h in range(N_DEV - 1):
                    r2 = ring_hop(g)
                    g += 1
                    origin = lax.rem(p + N_DEV - h - 1, N_DEV)
                    xfull[pl.ds(origin * M_PER, M_PER), :] = comm[r2, :, :]
            return g

        g = layer(win0_ref, wout0_ref, g, last=False)
        g = layer(win1_ref, wout1_ref, g, last=False)
        g = layer(win2_ref, wout2_ref, g, last=True)

    return pl.pallas_call(
        body,
        out_shape=jax.ShapeDtypeStruct((M_PER, D), jnp.bfloat16),
        in_specs=[pl.BlockSpec(memory_space=pltpu.VMEM)] * 7,
        out_specs=pl.BlockSpec(memory_space=pltpu.VMEM),
        scratch_shapes=[
            pltpu.VMEM((M, D), jnp.bfloat16),
            pltpu.VMEM((M, D), jnp.bfloat16),
            pltpu.VMEM((2, M_PER, D), jnp.bfloat16),
            pltpu.SemaphoreType.DMA((2,)),
            pltpu.SemaphoreType.DMA((2,)),
        ],
        compiler_params=pltpu.CompilerParams(collective_id=0),
    )(x, Win0, Wout0, Win1, Wout1, Win2, Wout2)


# device time: 45135 ns/iter; 1.3037x vs baseline; 1.3037x over previous
import jax
import jax.numpy as jnp
from jax import lax
from jax.experimental import pallas as pl
from jax.experimental.pallas import tpu as pltpu

N_DEV = 4
M_PER = 64
M = N_DEV * M_PER
D = 512
H_PER = 1024


def kernel(x, Win0, Wout0, Win1, Wout1, Win2, Wout2):
    def body(x_ref, win0_ref, wout0_ref, win1_ref, wout1_ref,
             win2_ref, wout2_ref, out_ref,
             xfull, zbuf, recvbuf, send_sems, recv_sems):
        p = lax.axis_index("i")
        a = p ^ 1
        b = 3 - p
        mychunk = p * M_PER
        abase = a * M_PER
        gbase = (p // 2) * (2 * M_PER)
        obase = (1 - p // 2) * (2 * M_PER)

        barrier_sem = pltpu.get_barrier_semaphore()
        for nbr in (a, b):
            pl.semaphore_signal(
                barrier_sem, inc=1,
                device_id=(nbr,), device_id_type=pl.DeviceIdType.MESH,
            )
        pl.semaphore_wait(barrier_sem, 2)

        def exchange(idx, partner, src, dst):
            rdma = pltpu.make_async_remote_copy(
                src_ref=src, dst_ref=dst,
                send_sem=send_sems.at[idx], recv_sem=recv_sems.at[idx],
                device_id=(partner,), device_id_type=pl.DeviceIdType.MESH,
            )
            rdma.start()
            return rdma

        xfull[pl.ds(mychunk, M_PER), :] = x_ref[:, :].astype(jnp.bfloat16)
        e0 = exchange(0, a,
                      xfull.at[pl.ds(mychunk, M_PER)],
                      xfull.at[pl.ds(mychunk, M_PER)])
        e0.wait()
        e1 = exchange(1, b,
                      xfull.at[pl.ds(gbase, 2 * M_PER)],
                      xfull.at[pl.ds(gbase, 2 * M_PER)])
        e1.wait()

        def compute_z(win_ref, wout_ref):
            X = xfull[:, :]
            W1 = win_ref[:, :].astype(jnp.bfloat16)
            W2 = wout_ref[:, :].astype(jnp.bfloat16)
            h1 = jnp.dot(X, W1, preferred_element_type=jnp.float32)
            h1 = jnp.maximum(h1, 0.0).astype(jnp.bfloat16)
            z = jnp.dot(h1, W2, preferred_element_type=jnp.float32)
            zbuf[:, :] = z.astype(jnp.bfloat16)

        def allreduce(sem0):
            e = exchange(sem0, b, zbuf.at[:], recvbuf.at[0])
            e.wait()
            zbuf[:, :] = zbuf[:, :] + recvbuf[0, :, :]
            e = exchange(sem0 + 1, a, zbuf.at[:], recvbuf.at[1])
            e.wait()
            xfull[:, :] = zbuf[:, :] + recvbuf[1, :, :]

        def reduce_scatter(sem0):
            e = exchange(sem0, b,
                         zbuf.at[pl.ds(obase, 2 * M_PER)],
                         recvbuf.at[0, pl.ds(obase, 2 * M_PER)])
            e.wait()
            zbuf[pl.ds(gbase, 2 * M_PER), :] = (
                zbuf[pl.ds(gbase, 2 * M_PER), :]
                + recvbuf[0, pl.ds(gbase, 2 * M_PER), :]
            )
            e = exchange(sem0 + 1, a,
                         zbuf.at[pl.ds(abase, M_PER)],
                         recvbuf.at[1, pl.ds(abase, M_PER)])
            e.wait()
            out_ref[:, :] = (
                zbuf[pl.ds(mychunk, M_PER), :]
                + recvbuf[1, pl.ds(mychunk, M_PER), :]
            )

        compute_z(win0_ref, wout0_ref)
        allreduce(2)
        compute_z(win1_ref, wout1_ref)
        allreduce(4)
        compute_z(win2_ref, wout2_ref)
        reduce_scatter(6)

    return pl.pallas_call(
        body,
        out_shape=jax.ShapeDtypeStruct((M_PER, D), jnp.bfloat16),
        in_specs=[pl.BlockSpec(memory_space=pltpu.VMEM)] * 7,
        out_specs=pl.BlockSpec(memory_space=pltpu.VMEM),
        scratch_shapes=[
            pltpu.VMEM((M, D), jnp.bfloat16),
            pltpu.VMEM((M, D), jnp.bfloat16),
            pltpu.VMEM((2, M, D), jnp.bfloat16),
            pltpu.SemaphoreType.DMA((8,)),
            pltpu.SemaphoreType.DMA((8,)),
        ],
        compiler_params=pltpu.CompilerParams(collective_id=0),
    )(x, Win0, Wout0, Win1, Wout1, Win2, Wout2)


# device time: 41863 ns/iter; 1.4056x vs baseline; 1.0782x over previous
import jax
import jax.numpy as jnp
from jax import lax
from jax.experimental import pallas as pl
from jax.experimental.pallas import tpu as pltpu

N_DEV = 4
M_PER = 64
M = N_DEV * M_PER
HALF = M // 2
D = 512
H_PER = 1024
BF = jnp.bfloat16


def kernel(x, Win0, Wout0, Win1, Wout1, Win2, Wout2):
    def body(x_ref, win0_ref, wout0_ref, win1_ref, wout1_ref,
             win2_ref, wout2_ref, out_ref,
             xfull, hbuf, zbuf, recvbuf, send_sems, recv_sems):
        p = lax.axis_index("i")
        a = p ^ 1
        b = 3 - p
        mychunk = p * M_PER
        abase = a * M_PER
        gbase = (p // 2) * HALF
        obase = (1 - p // 2) * HALF

        barrier_sem = pltpu.get_barrier_semaphore()
        for nbr in (a, b):
            pl.semaphore_signal(
                barrier_sem, inc=1,
                device_id=(nbr,), device_id_type=pl.DeviceIdType.MESH,
            )
        pl.semaphore_wait(barrier_sem, 2)

        def exchange(idx, partner, src, dst):
            rdma = pltpu.make_async_remote_copy(
                src_ref=src, dst_ref=dst,
                send_sem=send_sems.at[idx], recv_sem=recv_sems.at[idx],
                device_id=(partner,), device_id_type=pl.DeviceIdType.MESH,
            )
            rdma.start()
            return rdma

        xfull[pl.ds(mychunk, M_PER), :] = x_ref[:, :].astype(BF)
        e0 = exchange(0, a,
                      xfull.at[pl.ds(mychunk, M_PER)],
                      xfull.at[pl.ds(mychunk, M_PER)])
        e0.wait()
        e1 = exchange(1, b,
                      xfull.at[pl.ds(gbase, HALF)],
                      xfull.at[pl.ds(gbase, HALF)])
        W1 = win0_ref[:, :].astype(BF)
        hg = jnp.dot(xfull[pl.ds(gbase, HALF), :], W1,
                     preferred_element_type=jnp.float32)
        hbuf[pl.ds(gbase, HALF), :] = jnp.maximum(hg, 0.0).astype(BF)
        e1.wait()
        ho = jnp.dot(xfull[pl.ds(obase, HALF), :], W1,
                     preferred_element_type=jnp.float32)
        hbuf[pl.ds(obase, HALF), :] = jnp.maximum(ho, 0.0).astype(BF)

        def halfmm(src, r, w):
            return jnp.dot(src[pl.ds(r * HALF, HALF), :], w,
                           preferred_element_type=jnp.float32)

        def allreduce_fused(wout_ref, win_next_ref, sem0):
            W2 = wout_ref[:, :].astype(BF)
            zbuf[pl.ds(0, HALF), :] = halfmm(hbuf, 0, W2).astype(BF)
            eB0 = exchange(sem0, b,
                           zbuf.at[pl.ds(0, HALF)],
                           recvbuf.at[0, pl.ds(0, HALF)])
            zbuf[pl.ds(HALF, HALF), :] = halfmm(hbuf, 1, W2).astype(BF)
            eB1 = exchange(sem0 + 1, b,
                           zbuf.at[pl.ds(HALF, HALF)],
                           recvbuf.at[0, pl.ds(HALF, HALF)])
            eB0.wait()
            zbuf[pl.ds(0, HALF), :] = (
                zbuf[pl.ds(0, HALF), :] + recvbuf[0, pl.ds(0, HALF), :]
            )
            eA0 = exchange(sem0 + 2, a,
                           zbuf.at[pl.ds(0, HALF)],
                           recvbuf.at[1, pl.ds(0, HALF)])
            eB1.wait()
            zbuf[pl.ds(HALF, HALF), :] = (
                zbuf[pl.ds(HALF, HALF), :] + recvbuf[0, pl.ds(HALF, HALF), :]
            )
            eA1 = exchange(sem0 + 3, a,
                           zbuf.at[pl.ds(HALF, HALF)],
                           recvbuf.at[1, pl.ds(HALF, HALF)])
            W1n = win_next_ref[:, :].astype(BF)
            eA0.wait()
            xfull[pl.ds(0, HALF), :] = (
                zbuf[pl.ds(0, HALF), :] + recvbuf[1, pl.ds(0, HALF), :]
            )
            h0 = halfmm(xfull, 0, W1n)
            hbuf[pl.ds(0, HALF), :] = jnp.maximum(h0, 0.0).astype(BF)
            eA1.wait()
            xfull[pl.ds(HALF, HALF), :] = (
                zbuf[pl.ds(HALF, HALF), :] + recvbuf[1, pl.ds(HALF, HALF), :]
            )
            h1 = halfmm(xfull, 1, W1n)
            hbuf[pl.ds(HALF, HALF), :] = jnp.maximum(h1, 0.0).astype(BF)

        allreduce_fused(wout0_ref, win1_ref, 2)
        allreduce_fused(wout1_ref, win2_ref, 6)

        W2 = wout2_ref[:, :].astype(BF)
        zo = jnp.dot(hbuf[pl.ds(obase, HALF), :], W2,
                     preferred_element_type=jnp.float32)
        zbuf[pl.ds(obase, HALF), :] = zo.astype(BF)
        eRB = exchange(10, b,
                       zbuf.at[pl.ds(obase, HALF)],
                       recvbuf.at[0, pl.ds(obase, HALF)])
        zg = jnp.dot(hbuf[pl.ds(gbase, HALF), :], W2,
                     preferred_element_type=jnp.float32)
        zbuf[pl.ds(gbase, HALF), :] = zg.astype(BF)
        eRB.wait()
        zbuf[pl.ds(gbase, HALF), :] = (
            zbuf[pl.ds(gbase, HALF), :] + recvbuf[0, pl.ds(gbase, HALF), :]
        )
        eRA = exchange(11, a,
                       zbuf.at[pl.ds(abase, M_PER)],
                       recvbuf.at[1, pl.ds(abase, M_PER)])
        eRA.wait()
        out_ref[:, :] = (
            zbuf[pl.ds(mychunk, M_PER), :]
            + recvbuf[1, pl.ds(mychunk, M_PER), :]
        )

    return pl.pallas_call(
        body,
        out_shape=jax.ShapeDtypeStruct((M_PER, D), BF),
        in_specs=[pl.BlockSpec(memory_space=pltpu.VMEM)] * 7,
        out_specs=pl.BlockSpec(memory_space=pltpu.VMEM),
        scratch_shapes=[
            pltpu.VMEM((M, D), BF),
            pltpu.VMEM((M, H_PER), BF),
            pltpu.VMEM((M, D), BF),
            pltpu.VMEM((2, M, D), BF),
            pltpu.SemaphoreType.DMA((12,)),
            pltpu.SemaphoreType.DMA((12,)),
        ],
        compiler_params=pltpu.CompilerParams(collective_id=0),
    )(x, Win0, Wout0, Win1, Wout1, Win2, Wout2)
